# baseline (speedup 1.0000x reference)
import functools

import jax
import jax.numpy as jnp
from jax import lax
from jax.experimental import pallas as pl
from jax.experimental.pallas import tpu as pltpu
from jax.experimental.pallas import tpu_sc as plsc

N, D, K = 8192, 256, 1024
ITERS = 5
BM = 512
NBLK = N // BM
NW = 32
RPW = N // NW
HALF = RPW * D // 2
CHUNKS = HALF // 128


def _assign_body(x_ref, c_ref, labels_ref, counts_ref, idx_ref):
    i = pl.program_id(0)

    @pl.when(i == 0)
    def _():
        counts_ref[...] = jnp.zeros_like(counts_ref)

    x = x_ref[...]
    c = c_ref[...]
    x2 = jnp.sum(x * x, axis=1, keepdims=True)
    c2 = jnp.sum(c * c, axis=1)[None, :]
    d2 = x2 - 2.0 * jnp.dot(x, c.T) + c2
    lbl = jnp.argmin(d2, axis=1)
    labels_ref[0, 0, :] = lbl.astype(jnp.int32)
    onehot = (lbl[:, None] == jax.lax.broadcasted_iota(
        jnp.int32, (BM, K), 1)).astype(jnp.float32)
    counts_ref[0, :] += jnp.sum(onehot, axis=0)
    idx_ref[...] = lbl[:, None].astype(jnp.int32) * D + \
        jax.lax.broadcasted_iota(jnp.int32, (BM, D), 1)


def _assign(x, c):
    return pl.pallas_call(
        _assign_body,
        grid=(NBLK,),
        in_specs=[
            pl.BlockSpec((BM, D), lambda i: (i, 0)),
            pl.BlockSpec((K, D), lambda i: (0, 0)),
        ],
        out_specs=[
            pl.BlockSpec((1, 1, BM), lambda i: (i, 0, 0)),
            pl.BlockSpec((1, K), lambda i: (0, 0)),
            pl.BlockSpec((BM, D), lambda i: (i, 0)),
        ],
        out_shape=[
            jax.ShapeDtypeStruct((NBLK, 1, BM), jnp.int32),
            jax.ShapeDtypeStruct((1, K), jnp.float32),
            jax.ShapeDtypeStruct((N, D), jnp.int32),
        ],
    )(x, c)


def _segsum_sc_body(x_hbm, idx_hbm, zeros_hbm, out0_hbm, out1_hbm,
                    x_v, idx_v, acc_s, sem):
    ci = lax.axis_index("c")
    si = lax.axis_index("s")
    w = si * 2 + ci
    pltpu.sync_copy(zeros_hbm.at[pl.ds(si * 16384, 16384)],
                    acc_s.at[pl.ds(si * 16384, 16384)])

    def _half(h):
        pltpu.sync_copy(x_hbm.at[pl.ds(w * 2 * HALF + h * HALF, HALF)], x_v)
        pltpu.sync_copy(idx_hbm.at[pl.ds(w * 2 * CHUNKS + h * CHUNKS, CHUNKS)],
                        idx_v)
        if h == 0:
            plsc.subcore_barrier()
        def _fire(j, carry):
            pltpu.async_copy(x_v.at[pl.ds(j * 128, 128)],
                             acc_s.at[idx_v.at[j]], sem, add=True)
            return carry
        lax.fori_loop(0, CHUNKS, _fire, 0)
        pltpu.make_async_copy(x_hbm.at[pl.ds(0, HALF)], x_v, sem).wait()

    _half(0)
    _half(1)
    plsc.subcore_barrier()

    @pl.when(ci == 0)
    def _():
        pltpu.sync_copy(acc_s.at[pl.ds(si * 16384, 16384)],
                        out0_hbm.at[pl.ds(si * 16384, 16384)])

    @pl.when(ci == 1)
    def _():
        pltpu.sync_copy(acc_s.at[pl.ds(si * 16384, 16384)],
                        out1_hbm.at[pl.ds(si * 16384, 16384)])


_segsum_sc = functools.partial(
    pl.kernel,
    out_type=[jax.ShapeDtypeStruct((K * D,), jnp.float32),
              jax.ShapeDtypeStruct((K * D,), jnp.float32)],
    mesh=plsc.VectorSubcoreMesh(core_axis_name="c", subcore_axis_name="s"),
    scratch_types=[
        pltpu.VMEM((HALF,), jnp.float32),
        pltpu.VMEM((CHUNKS, 128), jnp.int32),
        pltpu.VMEM_SHARED((K * D,), jnp.float32),
        pltpu.SemaphoreType.DMA,
    ],
)(_segsum_sc_body)


def _update_body(p0_ref, p1_ref, counts_ref, c_ref, out_ref):
    p0 = p0_ref[...]
    p1 = p1_ref[...]
    counts = counts_ref[...]
    c = c_ref[...]
    sums = p0 + p1
    new_c = sums / jnp.maximum(counts, 1.0)
    out_ref[...] = jnp.where(counts > 0, new_c, c)


def _update(p0, p1, counts, c):
    return pl.pallas_call(
        _update_body,
        in_specs=[
            pl.BlockSpec((K, D), lambda: (0, 0)),
            pl.BlockSpec((K, D), lambda: (0, 0)),
            pl.BlockSpec((K, 1), lambda: (0, 0)),
            pl.BlockSpec((K, D), lambda: (0, 0)),
        ],
        out_specs=pl.BlockSpec((K, D), lambda: (0, 0)),
        out_shape=jax.ShapeDtypeStruct((K, D), jnp.float32),
    )(p0, p1, counts, c)


def kernel(x):
    x = x.reshape(x.shape[0], -1)
    x1d = x.reshape(-1)
    zeros = jnp.zeros((K * D,), jnp.float32)
    c = x[:K]
    for _ in range(ITERS):
        labels, counts, idx = _assign(x, c)
        p0, p1 = _segsum_sc(x1d, idx.reshape(N * D // 128, 128), zeros)
        c = _update(p0.reshape(K, D), p1.reshape(K, D),
                    counts.reshape(K, 1), c)
    labels, _, _ = _assign(x, c)
    return labels.reshape(N)
